# baseline (speedup 1.0000x reference)
import functools

import jax
import jax.numpy as jnp
from jax import lax
from jax.experimental import pallas as pl
from jax.experimental.pallas import tpu as pltpu
from jax.experimental.pallas import tpu_sc as plsc

NUM_EMBEDDINGS = 1000000
D = 32
PADDING_IDX = 0

_INFO = plsc.get_sparse_core_info()
NC = _INFO.num_cores
NS = _INFO.num_subcores
L = _INFO.num_lanes
NW = NC * NS

S = 200
BB = 16384
S_HI, S_LO = S // 8, 8
B_HI, B_LO = BB // 128, 128
C_HI, C_LO = D // 8, 8
K = S_LO * B_LO
Q = C_LO * B_LO

BH_PER_W = B_HI // NW
NSB = S_HI * BH_PER_W

_IGNORE = 7


@functools.partial(
    pl.kernel,
    out_type=(
        jax.ShapeDtypeStruct((S, C_HI, B_HI, Q), jnp.float32),
        jax.ShapeDtypeStruct((NC, D), jnp.float32),
    ),
    mesh=plsc.VectorSubcoreMesh(core_axis_name="c", subcore_axis_name="s"),
    scratch_types=[
        pltpu.VMEM((K,), jnp.int32),
        pltpu.VMEM((K,), jnp.int32),
        pltpu.VMEM((K,), jnp.int32),
        pltpu.VMEM((K,), jnp.int32),
        pltpu.VMEM((K, D), jnp.float32),
        pltpu.VMEM((K, D), jnp.float32),
        pltpu.VMEM((S_LO * C_HI * Q,), jnp.float32),
        pltpu.SemaphoreType.DMA,
        pltpu.SemaphoreType.DMA,
        pltpu.SemaphoreType.DMA,
        pltpu.SemaphoreType.DMA,
        pltpu.SemaphoreType.DMA,
    ],
    compiler_params=pltpu.CompilerParams(
        use_tc_tiling_on_sc=False, needs_layout_passes=False),
)
def _gather_kernel(idx3_hbm, table_hbm, out_hbm, zeros_hbm,
                   ibuf0, ibuf1, zbuf0, zbuf1, rows0, rows1, cbuf,
                   si0, si1, sg0, sg1, so):
    cid = lax.axis_index("c")
    sid = lax.axis_index("s")
    wid = sid * NC + cid

    ibufs, zbufs, rowss = (ibuf0, ibuf1), (zbuf0, zbuf1), (rows0, rows1)
    sis, sgs = (si0, si1), (sg0, sg1)

    def sb_coords(t):
        return t // BH_PER_W, wid * BH_PER_W + t % BH_PER_W

    def idx_start(t, b):
        s_hi, b_hi = sb_coords(t)
        pltpu.async_copy(idx3_hbm.at[s_hi, b_hi], ibufs[b], sis[b])

    def idx_wait(t, b):
        s_hi, b_hi = sb_coords(t)
        pltpu.make_async_copy(idx3_hbm.at[s_hi, b_hi], ibufs[b],
                              sis[b]).wait()

    def build_zbuf(b):
        ibuf, zbuf = ibufs[b], zbufs[b]

        def body(i, _):
            for k in range(4):
                sl = pl.ds((i * 4 + k) * L, L)
                v = ibuf[sl]
                zbuf[sl] = jnp.where(
                    v == PADDING_IDX, cid, _IGNORE).astype(jnp.int32)
            return 0

        lax.fori_loop(0, K // L // 4, body, 0)

    def gathers_start(b):
        pltpu.async_copy(
            table_hbm.at[plsc.Indices(ibufs[b], ignored_value=PADDING_IDX)],
            rowss[b], sgs[b])
        pltpu.async_copy(
            zeros_hbm.at[plsc.Indices(zbufs[b], ignored_value=_IGNORE)],
            rowss[b], sgs[b])

    def gathers_wait(b):
        pltpu.make_async_copy(
            table_hbm.at[plsc.Indices(ibufs[b], ignored_value=PADDING_IDX)],
            rowss[b], sgs[b]).wait()
        pltpu.make_async_copy(
            zeros_hbm.at[plsc.Indices(zbufs[b], ignored_value=_IGNORE)],
            rowss[b], sgs[b]).wait()

    def out_slices(t, i):
        s_hi, b_hi = sb_coords(t)
        s_lo, c_hi = i // C_HI, i % C_HI
        src = cbuf.at[pl.ds(i * Q, Q)]
        dst = out_hbm.at[s_hi * S_LO + s_lo, c_hi, b_hi]
        return src, dst

    def out_wait(t):
        def body(i, _):
            src, dst = out_slices(t, i)
            pltpu.make_async_copy(src, dst, so).wait()
            return 0

        lax.fori_loop(0, S_LO * C_HI, body, 0)

    def _pat(h):
        c = 16 * h + lax.iota(jnp.int32, L)
        return (c // C_LO) * Q + (c % C_LO) * B_LO

    def transpose_out(t, b):
        rows = rowss[b]
        pat0, pat1 = _pat(0), _pat(1)
        one = jnp.full((L,), 1, jnp.int32)

        def sbody(s_lo, _):
            base = s_lo * (C_HI * Q)
            d0 = pat0 + base
            d1 = pat1 + base

            def kbody(i, carry):
                c0, c1 = carry
                for u in range(4):
                    kk = s_lo * B_LO + i * 4 + u
                    plsc.store_scatter(cbuf, [c0], rows[kk, pl.ds(0, L)])
                    plsc.store_scatter(cbuf, [c1], rows[kk, pl.ds(L, L)])
                    c0 = c0 + one
                    c1 = c1 + one
                return c0, c1

            lax.fori_loop(0, B_LO // 4, kbody, (d0, d1))

            def obody(c_hi, _):
                src, dst = out_slices(t, s_lo * C_HI + c_hi)
                pltpu.async_copy(src, dst, so)
                return 0

            lax.fori_loop(0, C_HI, obody, 0)
            return 0

        lax.fori_loop(0, S_LO, sbody, 0)

    idx_start(0, 0)

    @pl.when(sid == 0)
    def _init_zero_row():
        zvec = jnp.zeros((L,), jnp.float32)
        for k in range(D // L):
            rows0[0, pl.ds(k * L, L)] = zvec
        pltpu.sync_copy(rows0.at[pl.ds(0, 1)], zeros_hbm.at[pl.ds(cid, 1)])

    plsc.subcore_barrier()

    idx_wait(0, 0)
    build_zbuf(0)
    gathers_start(0)
    idx_start(1, 1)

    def iteration(t, b):
        ob = 1 - b

        @pl.when(t < NSB - 1)
        def _prep_next():
            idx_wait(t + 1, ob)
            build_zbuf(ob)
            gathers_start(ob)

        gathers_wait(b)

        @pl.when(t < NSB - 2)
        def _prefetch_idx():
            idx_start(t + 2, b)

        @pl.when(t >= 1)
        def _drain_prev_out():
            out_wait(t - 1)

        transpose_out(t, b)
        return 0

    def loop_body(i, _):
        iteration(2 * i, 0)
        iteration(2 * i + 1, 1)
        return 0

    lax.fori_loop(0, NSB // 2, loop_body, 0)
    out_wait(NSB - 1)


def kernel(idx, embeddings):
    idx3 = jnp.transpose(
        jnp.transpose(idx, (1, 0)).reshape(S_HI, S_LO, B_HI, B_LO),
        (0, 2, 1, 3)).reshape(S_HI, B_HI, K).astype(jnp.int32)
    out4, _ = _gather_kernel(idx3, embeddings)
    return jnp.transpose(
        out4.reshape(S, C_HI, B_HI, C_LO, B_LO),
        (2, 4, 0, 1, 3)).reshape(BB, S, D)


# speedup vs baseline: 1.8032x; 1.8032x over previous
import functools

import jax
import jax.numpy as jnp
from jax import lax
from jax.experimental import pallas as pl
from jax.experimental.pallas import tpu as pltpu
from jax.experimental.pallas import tpu_sc as plsc

NUM_EMBEDDINGS = 1000000
D = 32
PADDING_IDX = 0

_INFO = plsc.get_sparse_core_info()
NC = _INFO.num_cores
NS = _INFO.num_subcores
L = _INFO.num_lanes
NW = NC * NS

S = 200
BB = 16384
S_HI, S_LO = S // 8, 8
B_HI, B_LO = BB // 128, 128
C_HI, C_LO = D // 8, 8
K = S_LO * B_LO
Q = C_LO * B_LO

BH_PER_W = B_HI // NW
NSB = S_HI * BH_PER_W

_IGNORE = 7


@functools.partial(
    pl.kernel,
    out_type=(
        jax.ShapeDtypeStruct((S, C_HI, B_HI, C_LO, B_LO), jnp.float32),
        jax.ShapeDtypeStruct((NC, D), jnp.float32),
    ),
    mesh=plsc.VectorSubcoreMesh(core_axis_name="c", subcore_axis_name="s"),
    scratch_types=[
        pltpu.VMEM((K,), jnp.int32),
        pltpu.VMEM((K,), jnp.int32),
        pltpu.VMEM((K,), jnp.int32),
        pltpu.VMEM((K,), jnp.int32),
        pltpu.VMEM((K, D), jnp.float32),
        pltpu.VMEM((K, D), jnp.float32),
        pltpu.VMEM((S_LO, C_HI, C_LO, B_LO + 1), jnp.float32),
        pltpu.SemaphoreType.DMA,
        pltpu.SemaphoreType.DMA,
        pltpu.SemaphoreType.DMA,
        pltpu.SemaphoreType.DMA,
        pltpu.SemaphoreType.DMA,
    ],
    compiler_params=pltpu.CompilerParams(
        use_tc_tiling_on_sc=False, needs_layout_passes=False,
        disable_bounds_checks=True),
)
def _gather_kernel(idx3_hbm, table_hbm, out_hbm, zeros_hbm,
                   ibuf0, ibuf1, zbuf0, zbuf1, rows0, rows1, cbuf,
                   si0, si1, sg0, sg1, so):
    cid = lax.axis_index("c")
    sid = lax.axis_index("s")
    wid = sid * NC + cid

    ibufs, zbufs, rowss = (ibuf0, ibuf1), (zbuf0, zbuf1), (rows0, rows1)
    sis, sgs = (si0, si1), (sg0, sg1)

    def sb_coords(t):
        return t // BH_PER_W, wid * BH_PER_W + t % BH_PER_W

    def idx_start(t, b):
        s_hi, b_hi = sb_coords(t)
        pltpu.async_copy(idx3_hbm.at[s_hi, b_hi], ibufs[b], sis[b])

    def idx_wait(t, b):
        s_hi, b_hi = sb_coords(t)
        pltpu.make_async_copy(idx3_hbm.at[s_hi, b_hi], ibufs[b],
                              sis[b]).wait()

    def build_zbuf(b):
        ibuf, zbuf = ibufs[b], zbufs[b]

        def body(i, _):
            for k in range(4):
                sl = pl.ds((i * 4 + k) * L, L)
                v = ibuf[sl]
                zbuf[sl] = jnp.where(
                    v == PADDING_IDX, cid, _IGNORE).astype(jnp.int32)
            return 0

        lax.fori_loop(0, K // L // 4, body, 0)

    def gathers_start(b):
        pltpu.async_copy(
            table_hbm.at[plsc.Indices(ibufs[b], ignored_value=PADDING_IDX)],
            rowss[b], sgs[b])
        pltpu.async_copy(
            zeros_hbm.at[plsc.Indices(zbufs[b], ignored_value=_IGNORE)],
            rowss[b], sgs[b])

    def gathers_wait(b):
        pltpu.make_async_copy(
            table_hbm.at[plsc.Indices(ibufs[b], ignored_value=PADDING_IDX)],
            rowss[b], sgs[b]).wait()
        pltpu.make_async_copy(
            zeros_hbm.at[plsc.Indices(zbufs[b], ignored_value=_IGNORE)],
            rowss[b], sgs[b]).wait()

    SK = B_LO + 1
    ST_CLO = SK
    ST_CHI = C_LO * SK
    ST_SLO = C_HI * ST_CHI

    def out_slices(t, s_lo):
        s_hi, b_hi = sb_coords(t)
        src = cbuf.at[s_lo, :, :, pl.ds(0, B_LO)]
        dst = out_hbm.at[s_hi * S_LO + s_lo, :, b_hi]
        return src, dst

    def out_wait(t):
        def body(s_lo, _):
            src, dst = out_slices(t, s_lo)
            pltpu.make_async_copy(src, dst, so).wait()
            return 0

        lax.fori_loop(0, S_LO, body, 0)

    def _pat(h):
        c = 16 * h + lax.iota(jnp.int32, L)
        return (c // C_LO) * ST_CHI + (c % C_LO) * ST_CLO

    def transpose_out(t, b):
        rows = rowss[b]
        pat0, pat1 = _pat(0), _pat(1)
        one = jnp.full((L,), 1, jnp.int32)
        zero = jnp.zeros((L,), jnp.int32)

        def sbody(s_lo, _):
            base = s_lo * ST_SLO
            d0 = pat0 + base
            d1 = pat1 + base

            def kbody(i, carry):
                c0, c1 = carry
                for u in range(4):
                    kk = s_lo * B_LO + i * 4 + u
                    plsc.store_scatter(cbuf, [zero, zero, zero, c0],
                                       rows[kk, pl.ds(0, L)])
                    plsc.store_scatter(cbuf, [zero, zero, zero, c1],
                                       rows[kk, pl.ds(L, L)])
                    c0 = c0 + one
                    c1 = c1 + one
                return c0, c1

            lax.fori_loop(0, B_LO // 4, kbody, (d0, d1))

            src, dst = out_slices(t, s_lo)
            pltpu.async_copy(src, dst, so)
            return 0

        lax.fori_loop(0, S_LO, sbody, 0)

    idx_start(0, 0)

    @pl.when(sid == 0)
    def _init_zero_row():
        zvec = jnp.zeros((L,), jnp.float32)
        for k in range(D // L):
            rows0[0, pl.ds(k * L, L)] = zvec
        pltpu.sync_copy(rows0.at[pl.ds(0, 1)], zeros_hbm.at[pl.ds(cid, 1)])

    plsc.subcore_barrier()

    idx_wait(0, 0)
    build_zbuf(0)
    gathers_start(0)
    idx_start(1, 1)

    def iteration(t, b):
        ob = 1 - b

        @pl.when(t < NSB - 1)
        def _prep_next():
            idx_wait(t + 1, ob)
            build_zbuf(ob)
            gathers_start(ob)

        gathers_wait(b)

        @pl.when(t < NSB - 2)
        def _prefetch_idx():
            idx_start(t + 2, b)

        @pl.when(t >= 1)
        def _drain_prev_out():
            out_wait(t - 1)

        transpose_out(t, b)
        return 0

    def loop_body(i, _):
        iteration(2 * i, 0)
        iteration(2 * i + 1, 1)
        return 0

    lax.fori_loop(0, NSB // 2, loop_body, 0)
    out_wait(NSB - 1)


def kernel(idx, embeddings):
    idx3 = jnp.transpose(
        jnp.transpose(idx, (1, 0)).reshape(S_HI, S_LO, B_HI, B_LO),
        (0, 2, 1, 3)).reshape(S_HI, B_HI, K).astype(jnp.int32)
    out5, _ = _gather_kernel(idx3, embeddings)
    return jnp.transpose(out5, (2, 4, 0, 1, 3)).reshape(BB, S, D)


# speedup vs baseline: 1.8121x; 1.0050x over previous
import functools

import jax
import jax.numpy as jnp
from jax import lax
from jax.experimental import pallas as pl
from jax.experimental.pallas import tpu as pltpu
from jax.experimental.pallas import tpu_sc as plsc

NUM_EMBEDDINGS = 1000000
D = 32
PADDING_IDX = 0

_INFO = plsc.get_sparse_core_info()
NC = _INFO.num_cores
NS = _INFO.num_subcores
L = _INFO.num_lanes
NW = NC * NS

S = 200
BB = 16384
S_HI, S_LO = S // 8, 8
B_HI, B_LO = BB // 128, 128
C_HI, C_LO = D // 8, 8
K = S_LO * B_LO
Q = C_LO * B_LO

BH_PER_W = B_HI // NW
NSB = S_HI * BH_PER_W

_IGNORE = 7


@functools.partial(
    pl.kernel,
    out_type=(
        jax.ShapeDtypeStruct((S, C_HI, B_HI, C_LO, B_LO), jnp.float32),
        jax.ShapeDtypeStruct((NC, D), jnp.float32),
    ),
    mesh=plsc.VectorSubcoreMesh(core_axis_name="c", subcore_axis_name="s"),
    scratch_types=[
        pltpu.VMEM((K,), jnp.int32),
        pltpu.VMEM((K,), jnp.int32),
        pltpu.VMEM((K,), jnp.int32),
        pltpu.VMEM((K,), jnp.int32),
        pltpu.VMEM((K, D), jnp.float32),
        pltpu.VMEM((K, D), jnp.float32),
        pltpu.VMEM((S_LO, C_HI, C_LO, B_LO + 1), jnp.float32),
        pltpu.SemaphoreType.DMA,
        pltpu.SemaphoreType.DMA,
        pltpu.SemaphoreType.DMA,
        pltpu.SemaphoreType.DMA,
        pltpu.SemaphoreType.DMA,
    ],
    compiler_params=pltpu.CompilerParams(
        use_tc_tiling_on_sc=False, needs_layout_passes=False,
        disable_bounds_checks=True),
)
def _gather_kernel(idx3_hbm, table_hbm, out_hbm, zeros_hbm,
                   ibuf0, ibuf1, zbuf0, zbuf1, rows0, rows1, cbuf,
                   si0, si1, sg0, sg1, so):
    cid = lax.axis_index("c")
    sid = lax.axis_index("s")
    wid = sid * NC + cid

    ibufs, zbufs, rowss = (ibuf0, ibuf1), (zbuf0, zbuf1), (rows0, rows1)
    sis, sgs = (si0, si1), (sg0, sg1)

    def sb_coords(t):
        return t // BH_PER_W, wid * BH_PER_W + t % BH_PER_W

    def idx_start(t, b):
        s_hi, b_hi = sb_coords(t)
        pltpu.async_copy(idx3_hbm.at[s_hi, b_hi], ibufs[b], sis[b])

    def idx_wait(t, b):
        s_hi, b_hi = sb_coords(t)
        pltpu.make_async_copy(idx3_hbm.at[s_hi, b_hi], ibufs[b],
                              sis[b]).wait()

    def build_zbuf(b):
        ibuf, zbuf = ibufs[b], zbufs[b]

        def body(i, _):
            for k in range(4):
                sl = pl.ds((i * 4 + k) * L, L)
                v = ibuf[sl]
                zbuf[sl] = jnp.where(
                    v == PADDING_IDX, cid, _IGNORE).astype(jnp.int32)
            return 0

        lax.fori_loop(0, K // L // 4, body, 0)

    def gathers_start(b):
        pltpu.async_copy(
            table_hbm.at[plsc.Indices(ibufs[b], ignored_value=PADDING_IDX)],
            rowss[b], sgs[b])
        pltpu.async_copy(
            zeros_hbm.at[plsc.Indices(zbufs[b], ignored_value=_IGNORE)],
            rowss[b], sgs[b])

    def gathers_wait(b):
        pltpu.make_async_copy(
            table_hbm.at[plsc.Indices(ibufs[b], ignored_value=PADDING_IDX)],
            rowss[b], sgs[b]).wait()
        pltpu.make_async_copy(
            zeros_hbm.at[plsc.Indices(zbufs[b], ignored_value=_IGNORE)],
            rowss[b], sgs[b]).wait()

    SK = B_LO + 1
    ST_CLO = SK
    ST_CHI = C_LO * SK
    ST_SLO = C_HI * ST_CHI

    def out_slices(t, s_lo):
        s_hi, b_hi = sb_coords(t)
        src = cbuf.at[s_lo, :, :, pl.ds(0, B_LO)]
        dst = out_hbm.at[s_hi * S_LO + s_lo, :, b_hi]
        return src, dst

    def out_wait(t):
        def body(s_lo, _):
            src, dst = out_slices(t, s_lo)
            pltpu.make_async_copy(src, dst, so).wait()
            return 0

        lax.fori_loop(0, S_LO, body, 0)

    def _chi(h):
        c = 16 * h + lax.iota(jnp.int32, L)
        return c // C_LO

    _clo = lax.iota(jnp.int32, L) % C_LO

    def transpose_out(t, b):
        rows = rowss[b]
        chi0, chi1 = _chi(0), _chi(1)
        one = jnp.full((L,), 1, jnp.int32)

        def sbody(s_lo, _):
            ssplat = jnp.full((L,), s_lo, jnp.int32)

            def kbody(i, bvec):
                for u in range(4):
                    kk = s_lo * B_LO + i * 4 + u
                    plsc.store_scatter(cbuf, [ssplat, chi0, _clo, bvec],
                                       rows[kk, pl.ds(0, L)])
                    plsc.store_scatter(cbuf, [ssplat, chi1, _clo, bvec],
                                       rows[kk, pl.ds(L, L)])
                    bvec = bvec + one
                return bvec

            lax.fori_loop(0, B_LO // 4, kbody, lax.iota(jnp.int32, L) * 0)

            src, dst = out_slices(t, s_lo)
            pltpu.async_copy(src, dst, so)
            return 0

        lax.fori_loop(0, S_LO, sbody, 0)

    idx_start(0, 0)

    @pl.when(sid == 0)
    def _init_zero_row():
        zvec = jnp.zeros((L,), jnp.float32)
        for k in range(D // L):
            rows0[0, pl.ds(k * L, L)] = zvec
        pltpu.sync_copy(rows0.at[pl.ds(0, 1)], zeros_hbm.at[pl.ds(cid, 1)])

    plsc.subcore_barrier()

    idx_wait(0, 0)
    build_zbuf(0)
    gathers_start(0)
    idx_start(1, 1)

    def iteration(t, b):
        ob = 1 - b

        @pl.when(t < NSB - 1)
        def _prep_next():
            idx_wait(t + 1, ob)
            build_zbuf(ob)
            gathers_start(ob)

        gathers_wait(b)

        @pl.when(t < NSB - 2)
        def _prefetch_idx():
            idx_start(t + 2, b)

        @pl.when(t >= 1)
        def _drain_prev_out():
            out_wait(t - 1)

        transpose_out(t, b)
        return 0

    def loop_body(i, _):
        iteration(2 * i, 0)
        iteration(2 * i + 1, 1)
        return 0

    lax.fori_loop(0, NSB // 2, loop_body, 0)
    out_wait(NSB - 1)


def kernel(idx, embeddings):
    idx3 = jnp.transpose(
        jnp.transpose(idx, (1, 0)).reshape(S_HI, S_LO, B_HI, B_LO),
        (0, 2, 1, 3)).reshape(S_HI, B_HI, K).astype(jnp.int32)
    out5, _ = _gather_kernel(idx3, embeddings)
    return jnp.transpose(out5, (2, 4, 0, 1, 3)).reshape(BB, S, D)


# speedup vs baseline: 1.8299x; 1.0098x over previous
import functools

import jax
import jax.numpy as jnp
from jax import lax
from jax.experimental import pallas as pl
from jax.experimental.pallas import tpu as pltpu
from jax.experimental.pallas import tpu_sc as plsc

NUM_EMBEDDINGS = 1000000
D = 32
PADDING_IDX = 0

_INFO = plsc.get_sparse_core_info()
NC = _INFO.num_cores
NS = _INFO.num_subcores
L = _INFO.num_lanes
NW = NC * NS

S = 200
BB = 16384
S_HI, S_LO = S // 8, 8
B_HI, B_LO = BB // 128, 128
C_HI, C_LO = D // 8, 8
K = S_LO * B_LO
Q = C_LO * B_LO

BH_PER_W = B_HI // NW
NSB = S_HI * BH_PER_W

_IGNORE = 7


@functools.partial(
    pl.kernel,
    out_type=(
        jax.ShapeDtypeStruct((S, C_HI, B_HI, C_LO, B_LO), jnp.float32),
        jax.ShapeDtypeStruct((NC, D), jnp.float32),
    ),
    mesh=plsc.VectorSubcoreMesh(core_axis_name="c", subcore_axis_name="s"),
    scratch_types=[
        pltpu.VMEM((K,), jnp.int32),
        pltpu.VMEM((K,), jnp.int32),
        pltpu.VMEM((K,), jnp.int32),
        pltpu.VMEM((K,), jnp.int32),
        pltpu.VMEM((K, D), jnp.float32),
        pltpu.VMEM((K, D), jnp.float32),
        pltpu.VMEM((S_LO, C_HI, C_LO, B_LO + 1), jnp.float32),
        pltpu.SemaphoreType.DMA,
        pltpu.SemaphoreType.DMA,
        pltpu.SemaphoreType.DMA,
        pltpu.SemaphoreType.DMA,
        pltpu.SemaphoreType.DMA,
    ],
    compiler_params=pltpu.CompilerParams(
        use_tc_tiling_on_sc=False, needs_layout_passes=False,
        disable_bounds_checks=True),
)
def _gather_kernel(idx3_hbm, table_hbm, out_hbm, zeros_hbm,
                   ibuf0, ibuf1, zbuf0, zbuf1, rows0, rows1, cbuf,
                   si0, si1, sg0, sg1, so):
    cid = lax.axis_index("c")
    sid = lax.axis_index("s")
    wid = sid * NC + cid

    ibufs, zbufs, rowss = (ibuf0, ibuf1), (zbuf0, zbuf1), (rows0, rows1)
    sis, sgs = (si0, si1), (sg0, sg1)

    def sb_coords(t):
        return t // BH_PER_W, wid * BH_PER_W + t % BH_PER_W

    def idx_start(t, b):
        s_hi, b_hi = sb_coords(t)
        pltpu.async_copy(idx3_hbm.at[s_hi, b_hi], ibufs[b], sis[b])

    def idx_wait(t, b):
        s_hi, b_hi = sb_coords(t)
        pltpu.make_async_copy(idx3_hbm.at[s_hi, b_hi], ibufs[b],
                              sis[b]).wait()

    def build_zbuf(b):
        ibuf, zbuf = ibufs[b], zbufs[b]

        def body(i, _):
            for k in range(4):
                sl = pl.ds((i * 4 + k) * L, L)
                v = ibuf[sl]
                zbuf[sl] = jnp.where(
                    v == PADDING_IDX, cid, _IGNORE).astype(jnp.int32)
            return 0

        lax.fori_loop(0, K // L // 4, body, 0)

    def gathers_start(b):
        pltpu.async_copy(
            table_hbm.at[plsc.Indices(ibufs[b], ignored_value=PADDING_IDX)],
            rowss[b], sgs[b])
        pltpu.async_copy(
            zeros_hbm.at[plsc.Indices(zbufs[b], ignored_value=_IGNORE)],
            rowss[b], sgs[b])

    def gathers_wait(b):
        pltpu.make_async_copy(
            table_hbm.at[plsc.Indices(ibufs[b], ignored_value=PADDING_IDX)],
            rowss[b], sgs[b]).wait()
        pltpu.make_async_copy(
            zeros_hbm.at[plsc.Indices(zbufs[b], ignored_value=_IGNORE)],
            rowss[b], sgs[b]).wait()

    SK = B_LO + 1
    ST_CLO = SK
    ST_CHI = C_LO * SK
    ST_SLO = C_HI * ST_CHI

    def out_slices(t, s_lo):
        s_hi, b_hi = sb_coords(t)
        src = cbuf.at[s_lo, :, :, pl.ds(0, B_LO)]
        dst = out_hbm.at[s_hi * S_LO + s_lo, :, b_hi]
        return src, dst

    def out_wait(t):
        def body(s_lo, _):
            src, dst = out_slices(t, s_lo)
            pltpu.make_async_copy(src, dst, so).wait()
            return 0

        lax.fori_loop(0, S_LO, body, 0)

    def _chi(h):
        c = 16 * h + lax.iota(jnp.int32, L)
        return c // C_LO

    _clo = lax.iota(jnp.int32, L) % C_LO

    def transpose_out(t, b):
        rows = rowss[b]
        chi0, chi1 = _chi(0), _chi(1)
        one = jnp.full((L,), 1, jnp.int32)

        def sbody(s_lo, _):
            ssplat = jnp.full((L,), s_lo, jnp.int32)

            def kbody(i, bvec):
                for u in range(8):
                    kk = s_lo * B_LO + i * 8 + u
                    plsc.store_scatter(cbuf, [ssplat, chi0, _clo, bvec],
                                       rows[kk, pl.ds(0, L)])
                    plsc.store_scatter(cbuf, [ssplat, chi1, _clo, bvec],
                                       rows[kk, pl.ds(L, L)])
                    bvec = bvec + one
                return bvec

            lax.fori_loop(0, B_LO // 8, kbody, lax.iota(jnp.int32, L) * 0)

            src, dst = out_slices(t, s_lo)
            pltpu.async_copy(src, dst, so)
            return 0

        lax.fori_loop(0, S_LO, sbody, 0)

    idx_start(0, 0)

    @pl.when(sid == 0)
    def _init_zero_row():
        zvec = jnp.zeros((L,), jnp.float32)
        for k in range(D // L):
            rows0[0, pl.ds(k * L, L)] = zvec
        pltpu.sync_copy(rows0.at[pl.ds(0, 1)], zeros_hbm.at[pl.ds(cid, 1)])

    plsc.subcore_barrier()

    idx_wait(0, 0)
    build_zbuf(0)
    gathers_start(0)
    idx_start(1, 1)

    def iteration(t, b):
        ob = 1 - b

        @pl.when(t < NSB - 1)
        def _prep_next():
            idx_wait(t + 1, ob)
            build_zbuf(ob)
            gathers_start(ob)

        gathers_wait(b)

        @pl.when(t < NSB - 2)
        def _prefetch_idx():
            idx_start(t + 2, b)

        @pl.when(t >= 1)
        def _drain_prev_out():
            out_wait(t - 1)

        transpose_out(t, b)
        return 0

    def loop_body(i, _):
        iteration(2 * i, 0)
        iteration(2 * i + 1, 1)
        return 0

    lax.fori_loop(0, NSB // 2, loop_body, 0)
    out_wait(NSB - 1)


def kernel(idx, embeddings):
    idx3 = jnp.transpose(
        jnp.transpose(idx, (1, 0)).reshape(S_HI, S_LO, B_HI, B_LO),
        (0, 2, 1, 3)).reshape(S_HI, B_HI, K).astype(jnp.int32)
    out5, _ = _gather_kernel(idx3, embeddings)
    return jnp.transpose(out5, (2, 4, 0, 1, 3)).reshape(BB, S, D)
